# Initial kernel scaffold; baseline (speedup 1.0000x reference)
import functools

import jax
import jax.numpy as jnp
from jax import lax
from jax.experimental import pallas as pl
from jax.experimental.pallas import tpu as pltpu

_N_E = 8192
_E_DIM = 64
_BETA = 0.25
_N_ROWS = 18432
_R = 512
_CCH = 2048
_NCH = _N_E // _CCH
_NT = _N_ROWS // _R


def _argmin_body(x_ref, xn_ref, c_ref, cn_ref, idx_ref, loss_ref, acc_ref):
    i = pl.program_id(0)
    xb = x_ref[...]
    xn = xn_ref[...]
    mv = jnp.full((_R, 1), jnp.inf, jnp.float32)
    mi = jnp.zeros((_R, 1), jnp.int32)
    for k in range(_NCH):
        cb = c_ref[k * _CCH:(k + 1) * _CCH, :]
        cnb = cn_ref[:, k * _CCH:(k + 1) * _CCH]
        xy = lax.dot_general(xb, cb, (((1,), (1,)), ((), ())),
                             preferred_element_type=jnp.float32)
        d = (xn + cnb) - 2.0 * xy
        cmin = jnp.min(d, axis=1, keepdims=True)
        ids = lax.broadcasted_iota(jnp.int32, (_R, _CCH), 1)
        lidx = jnp.min(jnp.where(d == cmin, ids, _CCH), axis=1, keepdims=True)
        gidx = lidx + k * _CCH
        better = cmin < mv
        mv = jnp.where(better, cmin, mv)
        mi = jnp.where(better, gidx, mi)
    idx_ref[...] = mi
    part = jnp.sum(mv)

    @pl.when(i == 0)
    def _():
        acc_ref[0, 0] = part

    @pl.when(i > 0)
    def _():
        acc_ref[0, 0] = acc_ref[0, 0] + part

    @pl.when(i == _NT - 1)
    def _():
        m = acc_ref[0, 0] / jnp.float32(_N_ROWS * _E_DIM)
        loss_ref[0, 0] = m + jnp.float32(_BETA) * m


def _distance_argmin(latent, x_norm, centroids, c_norm):
    return pl.pallas_call(
        _argmin_body,
        grid=(_NT,),
        in_specs=[
            pl.BlockSpec((_R, _E_DIM), lambda i: (i, 0)),
            pl.BlockSpec((_R, 1), lambda i: (i, 0)),
            pl.BlockSpec((_N_E, _E_DIM), lambda i: (0, 0)),
            pl.BlockSpec((1, _N_E), lambda i: (0, 0)),
        ],
        out_specs=[
            pl.BlockSpec((_R, 1), lambda i: (i, 0)),
            pl.BlockSpec((1, 1), lambda i: (0, 0)),
        ],
        out_shape=[
            jax.ShapeDtypeStruct((_N_ROWS, 1), jnp.int32),
            jax.ShapeDtypeStruct((1, 1), jnp.float32),
        ],
        scratch_shapes=[pltpu.SMEM((1, 1), jnp.float32)],
    )(latent, x_norm, centroids, c_norm)


def kernel(x, centroids):
    latent = x.reshape(-1, _E_DIM)
    x_norm = jnp.sum(latent ** 2, axis=1, keepdims=True)
    c_norm = jnp.sum(centroids ** 2, axis=1, keepdims=True)
    idx2, loss2 = _distance_argmin(latent, x_norm, centroids, c_norm.reshape(1, _N_E))
    indices = idx2.reshape(-1)
    x_q = jnp.take(centroids, indices, axis=0).reshape(x.shape)
    x_q_ste = x + (x_q - x)
    loss = loss2.reshape(())
    indices_out = indices.reshape(x.shape[:-1])
    return (x_q_ste, loss, indices_out)


# speedup vs baseline: 1.1351x; 1.1351x over previous
import functools

import jax
import jax.numpy as jnp
from jax import lax
from jax.experimental import pallas as pl
from jax.experimental.pallas import tpu as pltpu
from jax.experimental.pallas import tpu_sc as plsc

_N_E = 8192
_E_DIM = 64
_BETA = 0.25
_N_ROWS = 18432
_R = 512
_CCH = 2048
_NCH = _N_E // _CCH
_NT = _N_ROWS // _R


def _argmin_body(x_ref, xn_ref, c_ref, cn_ref, idx_ref, loss_ref, acc_ref):
    i = pl.program_id(0)
    xb = x_ref[...]
    xn = xn_ref[...]
    mv = jnp.full((_R, 1), jnp.inf, jnp.float32)
    mi = jnp.zeros((_R, 1), jnp.int32)
    for k in range(_NCH):
        cb = c_ref[k * _CCH:(k + 1) * _CCH, :]
        cnb = cn_ref[:, k * _CCH:(k + 1) * _CCH]
        xy = lax.dot_general(xb, cb, (((1,), (1,)), ((), ())),
                             preferred_element_type=jnp.float32)
        d = (xn + cnb) - 2.0 * xy
        cmin = jnp.min(d, axis=1, keepdims=True)
        ids = lax.broadcasted_iota(jnp.int32, (_R, _CCH), 1)
        lidx = jnp.min(jnp.where(d == cmin, ids, _CCH), axis=1, keepdims=True)
        gidx = lidx + k * _CCH
        better = cmin < mv
        mv = jnp.where(better, cmin, mv)
        mi = jnp.where(better, gidx, mi)
    idx_ref[...] = mi
    part = jnp.sum(mv)

    @pl.when(i == 0)
    def _():
        acc_ref[0, 0] = part

    @pl.when(i > 0)
    def _():
        acc_ref[0, 0] = acc_ref[0, 0] + part

    @pl.when(i == _NT - 1)
    def _():
        m = acc_ref[0, 0] / jnp.float32(_N_ROWS * _E_DIM)
        loss_ref[...] = jnp.full((1, 1), m + jnp.float32(_BETA) * m, jnp.float32)


def _distance_argmin(latent, x_norm, centroids, c_norm):
    return pl.pallas_call(
        _argmin_body,
        grid=(_NT,),
        in_specs=[
            pl.BlockSpec((_R, _E_DIM), lambda i: (i, 0)),
            pl.BlockSpec((_R, 1), lambda i: (i, 0)),
            pl.BlockSpec((_N_E, _E_DIM), lambda i: (0, 0)),
            pl.BlockSpec((1, _N_E), lambda i: (0, 0)),
        ],
        out_specs=[
            pl.BlockSpec((_R, 1), lambda i: (i, 0)),
            pl.BlockSpec((1, 1), lambda i: (0, 0)),
        ],
        out_shape=[
            jax.ShapeDtypeStruct((_N_ROWS, 1), jnp.int32),
            jax.ShapeDtypeStruct((1, 1), jnp.float32),
        ],
        scratch_shapes=[pltpu.SMEM((1, 1), jnp.float32)],
    )(latent, x_norm, centroids, c_norm)


_NW = 32
_BPW = _N_ROWS // _NW
_GCH = 96
_NG = _BPW // _GCH


def _make_gather_ste():
    mesh = plsc.VectorSubcoreMesh(core_axis_name="c", subcore_axis_name="s")

    @functools.partial(
        pl.kernel,
        mesh=mesh,
        out_type=jax.ShapeDtypeStruct((_N_ROWS * _E_DIM,), jnp.float32),
        scratch_types=[
            pltpu.VMEM((_BPW,), jnp.int32),
            pltpu.VMEM((_NG, _GCH, 128), jnp.float32),
            pltpu.VMEM((_BPW * _E_DIM,), jnp.float32),
            pltpu.SemaphoreType.DMA,
        ],
    )
    def gather_ste(cent_hbm, idx_hbm, x_hbm, out_hbm, idx_v, q_v, x_v, sem):
        wid = lax.axis_index("s") * 2 + lax.axis_index("c")
        base = wid * (_BPW * _E_DIM)
        pltpu.sync_copy(idx_hbm.at[pl.ds(wid * _BPW, _BPW)], idx_v)
        pltpu.sync_copy(x_hbm.at[pl.ds(base, _BPW * _E_DIM)], x_v)
        for g in range(_NG):
            pltpu.async_copy(
                cent_hbm.at[idx_v.at[pl.ds(g * _GCH, _GCH)]],
                q_v.at[g],
                sem,
            ).wait()

        def row(r, carry):
            for t in range(_E_DIM // 16):
                xq = q_v[r // _GCH, r % _GCH, pl.ds(t * 16, 16)]
                xx = x_v[pl.ds(r * _E_DIM + t * 16, 16)]
                x_v[pl.ds(r * _E_DIM + t * 16, 16)] = xx + (xq - xx)
            return carry

        lax.fori_loop(0, _BPW, row, 0)
        pltpu.sync_copy(x_v, out_hbm.at[pl.ds(base, _BPW * _E_DIM)])

    return gather_ste


def kernel(x, centroids):
    latent = x.reshape(-1, _E_DIM)
    x_norm = jnp.sum(latent ** 2, axis=1, keepdims=True)
    c_norm = jnp.sum(centroids ** 2, axis=1, keepdims=True)
    idx2, loss2 = _distance_argmin(latent, x_norm, centroids, c_norm.reshape(1, _N_E))
    indices = idx2.reshape(-1)
    cent_pad = jnp.pad(centroids, ((0, 0), (0, 128 - _E_DIM)))
    x_q_ste = _make_gather_ste()(cent_pad, indices, latent.reshape(-1)).reshape(x.shape)
    loss = loss2.reshape(())
    indices_out = indices.reshape(x.shape[:-1])
    return (x_q_ste, loss, indices_out)


# speedup vs baseline: 1.1865x; 1.0453x over previous
import functools

import jax
import jax.numpy as jnp
from jax import lax
from jax.experimental import pallas as pl
from jax.experimental.pallas import tpu as pltpu
from jax.experimental.pallas import tpu_sc as plsc

_N_E = 8192
_E_DIM = 64
_BETA = 0.25
_N_ROWS = 18432
_R = 1024
_CCH = 2048
_NCH = _N_E // _CCH
_NT = _N_ROWS // _R


def _argmin_body(x_ref, xn_ref, c_ref, cn_ref, idx_ref, loss_ref, acc_ref):
    i = pl.program_id(0)
    xb = x_ref[...]
    xn = xn_ref[...]
    mv = jnp.full((_R, 1), jnp.inf, jnp.float32)
    mi = jnp.zeros((_R, 1), jnp.int32)
    for k in range(_NCH):
        cb = c_ref[k * _CCH:(k + 1) * _CCH, :]
        cnb = cn_ref[:, k * _CCH:(k + 1) * _CCH]
        xy = lax.dot_general(xb, cb, (((1,), (1,)), ((), ())),
                             preferred_element_type=jnp.float32)
        d = (xn + cnb) - 2.0 * xy
        cmin = jnp.min(d, axis=1, keepdims=True)
        ids = lax.broadcasted_iota(jnp.int32, (_R, _CCH), 1)
        lidx = jnp.min(jnp.where(d == cmin, ids, _CCH), axis=1, keepdims=True)
        gidx = lidx + k * _CCH
        better = cmin < mv
        mv = jnp.where(better, cmin, mv)
        mi = jnp.where(better, gidx, mi)
    idx_ref[...] = mi
    part = jnp.sum(mv)

    @pl.when(i == 0)
    def _():
        acc_ref[0, 0] = part

    @pl.when(i > 0)
    def _():
        acc_ref[0, 0] = acc_ref[0, 0] + part

    @pl.when(i == _NT - 1)
    def _():
        m = acc_ref[0, 0] / jnp.float32(_N_ROWS * _E_DIM)
        loss_ref[...] = jnp.full((1, 1), m + jnp.float32(_BETA) * m, jnp.float32)


def _distance_argmin(latent, x_norm, centroids, c_norm):
    return pl.pallas_call(
        _argmin_body,
        grid=(_NT,),
        in_specs=[
            pl.BlockSpec((_R, _E_DIM), lambda i: (i, 0)),
            pl.BlockSpec((_R, 1), lambda i: (i, 0)),
            pl.BlockSpec((_N_E, _E_DIM), lambda i: (0, 0)),
            pl.BlockSpec((1, _N_E), lambda i: (0, 0)),
        ],
        out_specs=[
            pl.BlockSpec((_R, 1), lambda i: (i, 0)),
            pl.BlockSpec((1, 1), lambda i: (0, 0)),
        ],
        out_shape=[
            jax.ShapeDtypeStruct((_N_ROWS, 1), jnp.int32),
            jax.ShapeDtypeStruct((1, 1), jnp.float32),
        ],
        scratch_shapes=[pltpu.SMEM((1, 1), jnp.float32)],
    )(latent, x_norm, centroids, c_norm)


_NW = 32
_BPW = _N_ROWS // _NW
_GCH = 96
_NG = _BPW // _GCH


def _make_gather_ste():
    mesh = plsc.VectorSubcoreMesh(core_axis_name="c", subcore_axis_name="s")

    @functools.partial(
        pl.kernel,
        mesh=mesh,
        out_type=jax.ShapeDtypeStruct((_N_ROWS * _E_DIM,), jnp.float32),
        scratch_types=[
            pltpu.VMEM((_BPW,), jnp.int32),
            pltpu.VMEM((_NG, _GCH, 128), jnp.float32),
            pltpu.VMEM((_BPW * _E_DIM,), jnp.float32),
            pltpu.SemaphoreType.DMA,
        ],
    )
    def gather_ste(cent_hbm, idx_hbm, x_hbm, out_hbm, idx_v, q_v, x_v, sem):
        wid = lax.axis_index("s") * 2 + lax.axis_index("c")
        base = wid * (_BPW * _E_DIM)
        pltpu.sync_copy(idx_hbm.at[pl.ds(wid * _BPW, _BPW)], idx_v)
        pltpu.sync_copy(x_hbm.at[pl.ds(base, _BPW * _E_DIM)], x_v)
        for g in range(_NG):
            pltpu.async_copy(
                cent_hbm.at[idx_v.at[pl.ds(g * _GCH, _GCH)]],
                q_v.at[g],
                sem,
            ).wait()

        def row(r, carry):
            for t in range(_E_DIM // 16):
                xq = q_v[r // _GCH, r % _GCH, pl.ds(t * 16, 16)]
                xx = x_v[pl.ds(r * _E_DIM + t * 16, 16)]
                x_v[pl.ds(r * _E_DIM + t * 16, 16)] = xx + (xq - xx)
            return carry

        lax.fori_loop(0, _BPW, row, 0)
        pltpu.sync_copy(x_v, out_hbm.at[pl.ds(base, _BPW * _E_DIM)])

    return gather_ste


def kernel(x, centroids):
    latent = x.reshape(-1, _E_DIM)
    x_norm = jnp.sum(latent ** 2, axis=1, keepdims=True)
    c_norm = jnp.sum(centroids ** 2, axis=1, keepdims=True)
    idx2, loss2 = _distance_argmin(latent, x_norm, centroids, c_norm.reshape(1, _N_E))
    indices = idx2.reshape(-1)
    cent_pad = jnp.pad(centroids, ((0, 0), (0, 128 - _E_DIM)))
    x_q_ste = _make_gather_ste()(cent_pad, indices, latent.reshape(-1)).reshape(x.shape)
    loss = loss2.reshape(())
    indices_out = indices.reshape(x.shape[:-1])
    return (x_q_ste, loss, indices_out)


# speedup vs baseline: 1.2172x; 1.0259x over previous
import functools

import jax
import jax.numpy as jnp
from jax import lax
from jax.experimental import pallas as pl
from jax.experimental.pallas import tpu as pltpu
from jax.experimental.pallas import tpu_sc as plsc

_N_E = 8192
_E_DIM = 64
_BETA = 0.25
_N_ROWS = 18432
_R = 2048
_CCH = 2048
_NCH = _N_E // _CCH
_NT = _N_ROWS // _R


def _argmin_body(x_ref, xn_ref, c_ref, cn_ref, idx_ref, loss_ref, acc_ref):
    i = pl.program_id(0)
    xb = x_ref[...]
    xn = xn_ref[...]
    mv = jnp.full((_R, 1), jnp.inf, jnp.float32)
    mi = jnp.zeros((_R, 1), jnp.int32)
    for k in range(_NCH):
        cb = c_ref[k * _CCH:(k + 1) * _CCH, :]
        cnb = cn_ref[:, k * _CCH:(k + 1) * _CCH]
        xy = lax.dot_general(xb, cb, (((1,), (1,)), ((), ())),
                             preferred_element_type=jnp.float32)
        d = (xn + cnb) - 2.0 * xy
        cmin = jnp.min(d, axis=1, keepdims=True)
        ids = lax.broadcasted_iota(jnp.int32, (_R, _CCH), 1)
        lidx = jnp.min(jnp.where(d == cmin, ids, _CCH), axis=1, keepdims=True)
        gidx = lidx + k * _CCH
        better = cmin < mv
        mv = jnp.where(better, cmin, mv)
        mi = jnp.where(better, gidx, mi)
    idx_ref[...] = mi
    part = jnp.sum(mv)

    @pl.when(i == 0)
    def _():
        acc_ref[0, 0] = part

    @pl.when(i > 0)
    def _():
        acc_ref[0, 0] = acc_ref[0, 0] + part

    @pl.when(i == _NT - 1)
    def _():
        m = acc_ref[0, 0] / jnp.float32(_N_ROWS * _E_DIM)
        loss_ref[...] = jnp.full((1, 1), m + jnp.float32(_BETA) * m, jnp.float32)


def _distance_argmin(latent, x_norm, centroids, c_norm):
    return pl.pallas_call(
        _argmin_body,
        grid=(_NT,),
        in_specs=[
            pl.BlockSpec((_R, _E_DIM), lambda i: (i, 0)),
            pl.BlockSpec((_R, 1), lambda i: (i, 0)),
            pl.BlockSpec((_N_E, _E_DIM), lambda i: (0, 0)),
            pl.BlockSpec((1, _N_E), lambda i: (0, 0)),
        ],
        out_specs=[
            pl.BlockSpec((_R, 1), lambda i: (i, 0)),
            pl.BlockSpec((1, 1), lambda i: (0, 0)),
        ],
        out_shape=[
            jax.ShapeDtypeStruct((_N_ROWS, 1), jnp.int32),
            jax.ShapeDtypeStruct((1, 1), jnp.float32),
        ],
        scratch_shapes=[pltpu.SMEM((1, 1), jnp.float32)],
    )(latent, x_norm, centroids, c_norm)


_NW = 32
_BPW = _N_ROWS // _NW
_GCH = 96
_NG = _BPW // _GCH


def _make_gather_ste():
    mesh = plsc.VectorSubcoreMesh(core_axis_name="c", subcore_axis_name="s")

    @functools.partial(
        pl.kernel,
        mesh=mesh,
        out_type=jax.ShapeDtypeStruct((_N_ROWS * _E_DIM,), jnp.float32),
        scratch_types=[
            pltpu.VMEM((_BPW,), jnp.int32),
            pltpu.VMEM((_NG, _GCH, 128), jnp.float32),
            pltpu.VMEM((_BPW * _E_DIM,), jnp.float32),
            pltpu.SemaphoreType.DMA,
        ],
    )
    def gather_ste(cent_hbm, idx_hbm, x_hbm, out_hbm, idx_v, q_v, x_v, sem):
        wid = lax.axis_index("s") * 2 + lax.axis_index("c")
        base = wid * (_BPW * _E_DIM)
        pltpu.sync_copy(idx_hbm.at[pl.ds(wid * _BPW, _BPW)], idx_v)
        pltpu.sync_copy(x_hbm.at[pl.ds(base, _BPW * _E_DIM)], x_v)
        for g in range(_NG):
            pltpu.async_copy(
                cent_hbm.at[idx_v.at[pl.ds(g * _GCH, _GCH)]],
                q_v.at[g],
                sem,
            ).wait()

        def row(r, carry):
            for t in range(_E_DIM // 16):
                xq = q_v[r // _GCH, r % _GCH, pl.ds(t * 16, 16)]
                xx = x_v[pl.ds(r * _E_DIM + t * 16, 16)]
                x_v[pl.ds(r * _E_DIM + t * 16, 16)] = xx + (xq - xx)
            return carry

        lax.fori_loop(0, _BPW, row, 0)
        pltpu.sync_copy(x_v, out_hbm.at[pl.ds(base, _BPW * _E_DIM)])

    return gather_ste


def kernel(x, centroids):
    latent = x.reshape(-1, _E_DIM)
    x_norm = jnp.sum(latent ** 2, axis=1, keepdims=True)
    c_norm = jnp.sum(centroids ** 2, axis=1, keepdims=True)
    idx2, loss2 = _distance_argmin(latent, x_norm, centroids, c_norm.reshape(1, _N_E))
    indices = idx2.reshape(-1)
    cent_pad = jnp.pad(centroids, ((0, 0), (0, 128 - _E_DIM)))
    x_q_ste = _make_gather_ste()(cent_pad, indices, latent.reshape(-1)).reshape(x.shape)
    loss = loss2.reshape(())
    indices_out = indices.reshape(x.shape[:-1])
    return (x_q_ste, loss, indices_out)
